# initial kernel scaffold (unmeasured)
import functools

import jax
import jax.numpy as jnp
from jax import lax
from jax.experimental import pallas as pl
from jax.experimental.pallas import tpu as pltpu

N_DEV = 4


def kernel(x, w_mat):
    m, k = x.shape
    _, n = w_mat.shape
    ch = m // N_DEV
    nh = n // 2

    def body(x_ref, w_ref, out_ref, rbuf_a, rbuf_b,
             send_a, recv_a, send_b, recv_b):
        p = lax.axis_index("i")
        right = jnp.mod(p + 1, N_DEV)
        left = jnp.mod(p - 1, N_DEV)

        barrier = pltpu.get_barrier_semaphore()
        for nbr in (left, right):
            pl.semaphore_signal(barrier, inc=1, device_id=(nbr,),
                                device_id_type=pl.DeviceIdType.MESH)
        pl.semaphore_wait(barrier, 2)

        for c in range(N_DEV):
            rows = pl.ds(c * ch, ch)
            out_ref[rows, :] = jnp.dot(x_ref[rows, :], w_ref[:, :],
                                       preferred_element_type=jnp.float32)

        cols_a = pl.ds(0, nh)
        cols_b = pl.ds(nh, nh)

        for s in range(N_DEV - 1):
            cs_a = jnp.mod(p - s, N_DEV)
            cs_b = jnp.mod(p + s, N_DEV)
            rdma_a = pltpu.make_async_remote_copy(
                src_ref=out_ref.at[pl.ds(cs_a * ch, ch), cols_a],
                dst_ref=rbuf_a.at[s],
                send_sem=send_a.at[s], recv_sem=recv_a.at[s],
                device_id=(right,), device_id_type=pl.DeviceIdType.MESH)
            rdma_b = pltpu.make_async_remote_copy(
                src_ref=out_ref.at[pl.ds(cs_b * ch, ch), cols_b],
                dst_ref=rbuf_b.at[s],
                send_sem=send_b.at[s], recv_sem=recv_b.at[s],
                device_id=(left,), device_id_type=pl.DeviceIdType.MESH)
            rdma_a.start()
            rdma_b.start()
            rdma_a.wait()
            rdma_b.wait()
            cr_a = jnp.mod(p - s - 1, N_DEV)
            cr_b = jnp.mod(p + s + 1, N_DEV)
            rows_a = pl.ds(cr_a * ch, ch)
            rows_b = pl.ds(cr_b * ch, ch)
            out_ref[rows_a, cols_a] = out_ref[rows_a, cols_a] + rbuf_a[s]
            out_ref[rows_b, cols_b] = out_ref[rows_b, cols_b] + rbuf_b[s]

        for s in range(N_DEV - 1):
            ca = jnp.mod(p + 1 - s, N_DEV)
            cb = jnp.mod(p - 1 + s, N_DEV)
            rows_a = pl.ds(ca * ch, ch)
            rows_b = pl.ds(cb * ch, ch)
            rdma_a = pltpu.make_async_remote_copy(
                src_ref=out_ref.at[rows_a, cols_a],
                dst_ref=out_ref.at[rows_a, cols_a],
                send_sem=send_a.at[3 + s], recv_sem=recv_a.at[3 + s],
                device_id=(right,), device_id_type=pl.DeviceIdType.MESH)
            rdma_b = pltpu.make_async_remote_copy(
                src_ref=out_ref.at[rows_b, cols_b],
                dst_ref=out_ref.at[rows_b, cols_b],
                send_sem=send_b.at[3 + s], recv_sem=recv_b.at[3 + s],
                device_id=(left,), device_id_type=pl.DeviceIdType.MESH)
            rdma_a.start()
            rdma_b.start()
            rdma_a.wait()
            rdma_b.wait()

        amax = jnp.float32(0.0)
        for c in range(N_DEV):
            rows = pl.ds(c * ch, ch)
            amax = jnp.maximum(amax, jnp.max(jnp.maximum(out_ref[rows, :], 0.0)))
        scale = amax / 127.0
        for c in range(N_DEV):
            rows = pl.ds(c * ch, ch)
            y = jnp.maximum(out_ref[rows, :], 0.0)
            q = jnp.clip(jnp.round(y / scale), -127.0, 127.0)
            out_ref[rows, :] = q * scale

        @functools.partial(pl.run_scoped,
                           exit_sem=pltpu.SemaphoreType.REGULAR)
        def _(exit_sem):
            for nbr in (left, right):
                pl.semaphore_signal(exit_sem, inc=1, device_id=(nbr,),
                                    device_id_type=pl.DeviceIdType.MESH)
            pl.semaphore_wait(exit_sem, 2)

    return pl.pallas_call(
        body,
        out_shape=jax.ShapeDtypeStruct((m, n), jnp.float32),
        in_specs=[pl.BlockSpec(memory_space=pltpu.VMEM),
                  pl.BlockSpec(memory_space=pltpu.VMEM)],
        out_specs=pl.BlockSpec(memory_space=pltpu.VMEM),
        scratch_shapes=[
            pltpu.VMEM((N_DEV - 1, ch, nh), jnp.float32),
            pltpu.VMEM((N_DEV - 1, ch, nh), jnp.float32),
            pltpu.SemaphoreType.DMA((6,)),
            pltpu.SemaphoreType.DMA((6,)),
            pltpu.SemaphoreType.DMA((6,)),
            pltpu.SemaphoreType.DMA((6,)),
        ],
        compiler_params=pltpu.CompilerParams(collective_id=0),
    )(x, w_mat)


# baseline (device time: 342098 ns/iter reference)
import functools

import jax
import jax.numpy as jnp
from jax import lax
from jax.experimental import pallas as pl
from jax.experimental.pallas import tpu as pltpu

N_DEV = 4


def kernel(x, w_mat):
    m, k = x.shape
    _, n = w_mat.shape
    ch = m // N_DEV
    nh = n // 2

    def body(x_hbm, w_ref, out_hbm, xbuf, buf_a, buf_b, amax_buf,
             rs_send_a, rs_recv_a, rs_send_b, rs_recv_b,
             ag_send_a, ag_recv_a, ag_send_b, ag_recv_b,
             ax_send, ax_recv, xsem, osem):
        p = lax.axis_index("i")
        right = jnp.mod(p + 1, N_DEV)
        left = jnp.mod(p - 1, N_DEV)
        cols_a = pl.ds(0, nh)
        cols_b = pl.ds(nh, nh)

        barrier = pltpu.get_barrier_semaphore()
        for nbr in (left, right):
            pl.semaphore_signal(barrier, inc=1, device_id=(nbr,),
                                device_id_type=pl.DeviceIdType.MESH)
        pl.semaphore_wait(barrier, 2)

        def load_x(c, slot):
            cp = pltpu.make_async_copy(
                x_hbm.at[pl.ds(c * ch, ch), :], xbuf.at[slot],
                xsem.at[slot])
            cp.start()
            return cp

        load_x(p, 0).wait()
        buf_a[0, :, :] = jnp.dot(xbuf[0, :, :], w_ref[:, :nh],
                                 preferred_element_type=jnp.float32)
        buf_b[0, :, :] = jnp.dot(xbuf[0, :, :], w_ref[:, nh:],
                                 preferred_element_type=jnp.float32)

        for s in range(N_DEV - 1):
            rdma_a = pltpu.make_async_remote_copy(
                src_ref=buf_a.at[s], dst_ref=buf_a.at[s + 1],
                send_sem=rs_send_a.at[s], recv_sem=rs_recv_a.at[s],
                device_id=(right,), device_id_type=pl.DeviceIdType.MESH)
            rdma_b = pltpu.make_async_remote_copy(
                src_ref=buf_b.at[s], dst_ref=buf_b.at[s + 1],
                send_sem=rs_send_b.at[s], recv_sem=rs_recv_b.at[s],
                device_id=(left,), device_id_type=pl.DeviceIdType.MESH)
            rdma_a.start()
            rdma_b.start()
            ca = jnp.mod(p - 1 - s, N_DEV)
            cb = jnp.mod(p + 1 + s, N_DEV)
            if s == 1:
                cpa = load_x(ca, 0)
                cpb = None
            else:
                cpa = load_x(ca, 0)
                cpb = load_x(cb, 1)
            rdma_a.wait()
            rdma_b.wait()
            cpa.wait()
            buf_a[s + 1, :, :] = buf_a[s + 1, :, :] + jnp.dot(
                xbuf[0, :, :], w_ref[:, :nh],
                preferred_element_type=jnp.float32)
            b_slot = 0
            if cpb is not None:
                cpb.wait()
                b_slot = 1
            buf_b[s + 1, :, :] = buf_b[s + 1, :, :] + jnp.dot(
                xbuf[b_slot, :, :], w_ref[:, nh:],
                preferred_element_type=jnp.float32)


        buf_a[3, :, :] = jnp.maximum(buf_a[3, :, :], 0.0)
        buf_b[3, :, :] = jnp.maximum(buf_b[3, :, :], 0.0)
        local_amax = jnp.maximum(jnp.max(buf_a[3, :, :]),
                                 jnp.max(buf_b[3, :, :]))
        amax_buf[pl.ds(p, 1), :, :] = jnp.full((1, 8, 128), local_amax,
                                               jnp.float32)

        ax_rdmas = []
        for j in range(1, N_DEV):
            tgt = jnp.mod(p + j, N_DEV)
            r = pltpu.make_async_remote_copy(
                src_ref=amax_buf.at[pl.ds(p, 1)],
                dst_ref=amax_buf.at[pl.ds(p, 1)],
                send_sem=ax_send.at[j - 1], recv_sem=ax_recv.at[j - 1],
                device_id=(tgt,), device_id_type=pl.DeviceIdType.MESH)
            r.start()
            ax_rdmas.append(r)
        for r in ax_rdmas:
            r.wait()
        amax = jnp.max(amax_buf[:, :, :])
        scale = amax / 127.0

        buf_a[3, :, :] = jnp.clip(jnp.round(buf_a[3, :, :] / scale),
                                  -127.0, 127.0) * scale
        buf_b[3, :, :] = jnp.clip(jnp.round(buf_b[3, :, :] / scale),
                                  -127.0, 127.0) * scale

        oa = pltpu.make_async_copy(
            buf_a.at[3],
            out_hbm.at[pl.ds(jnp.mod(p + 1, N_DEV) * ch, ch), cols_a],
            osem.at[0])
        ob = pltpu.make_async_copy(
            buf_b.at[3],
            out_hbm.at[pl.ds(jnp.mod(p - 1, N_DEV) * ch, ch), cols_b],
            osem.at[1])
        oa.start()
        ob.start()
        oa.wait()
        ob.wait()

        for s in range(N_DEV - 1):
            ca = jnp.mod(p + 1 - s, N_DEV)
            cb = jnp.mod(p - 1 + s, N_DEV)
            rows_a = pl.ds(ca * ch, ch)
            rows_b = pl.ds(cb * ch, ch)
            rdma_a = pltpu.make_async_remote_copy(
                src_ref=out_hbm.at[rows_a, cols_a],
                dst_ref=out_hbm.at[rows_a, cols_a],
                send_sem=ag_send_a.at[s], recv_sem=ag_recv_a.at[s],
                device_id=(right,), device_id_type=pl.DeviceIdType.MESH)
            rdma_b = pltpu.make_async_remote_copy(
                src_ref=out_hbm.at[rows_b, cols_b],
                dst_ref=out_hbm.at[rows_b, cols_b],
                send_sem=ag_send_b.at[s], recv_sem=ag_recv_b.at[s],
                device_id=(left,), device_id_type=pl.DeviceIdType.MESH)
            rdma_a.start()
            rdma_b.start()
            rdma_a.wait()
            rdma_b.wait()

        @functools.partial(pl.run_scoped,
                           exit_sem=pltpu.SemaphoreType.REGULAR)
        def _(exit_sem):
            for nbr in (left, right):
                pl.semaphore_signal(exit_sem, inc=1, device_id=(nbr,),
                                    device_id_type=pl.DeviceIdType.MESH)
            pl.semaphore_wait(exit_sem, 2)

    return pl.pallas_call(
        body,
        out_shape=jax.ShapeDtypeStruct((m, n), jnp.float32),
        in_specs=[pl.BlockSpec(memory_space=pl.ANY),
                  pl.BlockSpec(memory_space=pltpu.VMEM)],
        out_specs=pl.BlockSpec(memory_space=pl.ANY),
        scratch_shapes=[
            pltpu.VMEM((2, ch, k), jnp.float32),
            pltpu.VMEM((N_DEV, ch, nh), jnp.float32),
            pltpu.VMEM((N_DEV, ch, nh), jnp.float32),
            pltpu.VMEM((N_DEV, 8, 128), jnp.float32),
            pltpu.SemaphoreType.DMA((3,)),
            pltpu.SemaphoreType.DMA((3,)),
            pltpu.SemaphoreType.DMA((3,)),
            pltpu.SemaphoreType.DMA((3,)),
            pltpu.SemaphoreType.DMA((3,)),
            pltpu.SemaphoreType.DMA((3,)),
            pltpu.SemaphoreType.DMA((3,)),
            pltpu.SemaphoreType.DMA((3,)),
            pltpu.SemaphoreType.DMA((3,)),
            pltpu.SemaphoreType.DMA((3,)),
            pltpu.SemaphoreType.DMA((2,)),
            pltpu.SemaphoreType.DMA((2,)),
        ],
        compiler_params=pltpu.CompilerParams(
            collective_id=0,
            vmem_limit_bytes=60 * 1024 * 1024,
        ),
    )(x, w_mat)


# device time: 244777 ns/iter; 1.3976x vs baseline; 1.3976x over previous
import functools

import jax
import jax.numpy as jnp
from jax import lax
from jax.experimental import pallas as pl
from jax.experimental.pallas import tpu as pltpu

N_DEV = 4


def kernel(x, w_mat):
    m, k = x.shape
    _, n = w_mat.shape
    ch = m // N_DEV
    nh = n // 2

    def body(x_hbm, w_ref, out_hbm, xbuf, buf_a, buf_b, qbuf_a, qbuf_b,
             amax_buf, rs_send_a, rs_recv_a, rs_send_b, rs_recv_b,
             ag_send_a, ag_recv_a, ag_send_b, ag_recv_b,
             ax_send, ax_recv, xsem, o_a, o_b):
        p = lax.axis_index("i")
        right = jnp.mod(p + 1, N_DEV)
        left = jnp.mod(p - 1, N_DEV)
        cols_a = pl.ds(0, nh)
        cols_b = pl.ds(nh, nh)

        barrier = pltpu.get_barrier_semaphore()
        for nbr in (left, right):
            pl.semaphore_signal(barrier, inc=1, device_id=(nbr,),
                                device_id_type=pl.DeviceIdType.MESH)
        pl.semaphore_wait(barrier, 2)

        def load_x(c):
            cp = pltpu.make_async_copy(
                x_hbm.at[pl.ds(c * ch, ch), :], xbuf.at[0], xsem)
            cp.start()
            return cp

        load_x(p).wait()
        buf_a[0, :, :] = jnp.dot(xbuf[0, :, :], w_ref[:, :nh],
                                 preferred_element_type=jnp.float32)
        buf_b[0, :, :] = jnp.dot(xbuf[0, :, :], w_ref[:, nh:],
                                 preferred_element_type=jnp.float32)

        for s in range(N_DEV - 1):
            rdma_a = pltpu.make_async_remote_copy(
                src_ref=buf_a.at[s], dst_ref=buf_a.at[s + 1],
                send_sem=rs_send_a.at[s], recv_sem=rs_recv_a.at[s],
                device_id=(right,), device_id_type=pl.DeviceIdType.MESH)
            rdma_b = pltpu.make_async_remote_copy(
                src_ref=buf_b.at[s], dst_ref=buf_b.at[s + 1],
                send_sem=rs_send_b.at[s], recv_sem=rs_recv_b.at[s],
                device_id=(left,), device_id_type=pl.DeviceIdType.MESH)
            rdma_a.start()
            rdma_b.start()
            ca = jnp.mod(p - 1 - s, N_DEV)
            cb = jnp.mod(p + 1 + s, N_DEV)
            load_x(ca).wait()
            rdma_a.wait()
            buf_a[s + 1, :, :] = buf_a[s + 1, :, :] + jnp.dot(
                xbuf[0, :, :], w_ref[:, :nh],
                preferred_element_type=jnp.float32)
            if s != 1:
                load_x(cb).wait()
            rdma_b.wait()
            buf_b[s + 1, :, :] = buf_b[s + 1, :, :] + jnp.dot(
                xbuf[0, :, :], w_ref[:, nh:],
                preferred_element_type=jnp.float32)


        buf_a[3, :, :] = jnp.maximum(buf_a[3, :, :], 0.0)
        buf_b[3, :, :] = jnp.maximum(buf_b[3, :, :], 0.0)
        local_amax = jnp.maximum(jnp.max(buf_a[3, :, :]),
                                 jnp.max(buf_b[3, :, :]))
        amax_buf[pl.ds(p, 1), :, :] = jnp.full((1, 8, 128), local_amax,
                                               jnp.float32)

        ax_rdmas = []
        for j in range(1, N_DEV):
            tgt = jnp.mod(p + j, N_DEV)
            r = pltpu.make_async_remote_copy(
                src_ref=amax_buf.at[pl.ds(p, 1)],
                dst_ref=amax_buf.at[pl.ds(p, 1)],
                send_sem=ax_send.at[j - 1], recv_sem=ax_recv.at[j - 1],
                device_id=(tgt,), device_id_type=pl.DeviceIdType.MESH)
            r.start()
            ax_rdmas.append(r)
        for r in ax_rdmas:
            r.wait()
        amax = jnp.max(amax_buf[:, :, :])
        scale = amax / 127.0

        qbuf_a[0, :, :] = jnp.clip(jnp.round(buf_a[3, :, :] / scale),
                                   -127.0, 127.0).astype(jnp.int8)
        qbuf_b[0, :, :] = jnp.clip(jnp.round(buf_b[3, :, :] / scale),
                                   -127.0, 127.0).astype(jnp.int8)

        out_copies = []

        def stage_out(slot):
            buf_a[slot, :, :] = (qbuf_a[slot, :, :].astype(jnp.float32)
                                 * scale)
            ra = jnp.mod(p + 1 - slot, N_DEV)
            oa = pltpu.make_async_copy(
                buf_a.at[slot],
                out_hbm.at[pl.ds(ra * ch, ch), cols_a], o_a.at[slot])
            oa.start()
            buf_b[slot, :, :] = (qbuf_b[slot, :, :].astype(jnp.float32)
                                 * scale)
            rb = jnp.mod(p - 1 + slot, N_DEV)
            ob = pltpu.make_async_copy(
                buf_b.at[slot],
                out_hbm.at[pl.ds(rb * ch, ch), cols_b], o_b.at[slot])
            ob.start()
            out_copies.extend((oa, ob))

        for s in range(N_DEV - 1):
            rdma_a = pltpu.make_async_remote_copy(
                src_ref=qbuf_a.at[s], dst_ref=qbuf_a.at[s + 1],
                send_sem=ag_send_a.at[s], recv_sem=ag_recv_a.at[s],
                device_id=(right,), device_id_type=pl.DeviceIdType.MESH)
            rdma_b = pltpu.make_async_remote_copy(
                src_ref=qbuf_b.at[s], dst_ref=qbuf_b.at[s + 1],
                send_sem=ag_send_b.at[s], recv_sem=ag_recv_b.at[s],
                device_id=(left,), device_id_type=pl.DeviceIdType.MESH)
            rdma_a.start()
            rdma_b.start()
            stage_out(s)
            rdma_a.wait()
            rdma_b.wait()
        stage_out(N_DEV - 1)

        for cp in out_copies:
            cp.wait()

        @functools.partial(pl.run_scoped,
                           exit_sem=pltpu.SemaphoreType.REGULAR)
        def _(exit_sem):
            for nbr in (left, right):
                pl.semaphore_signal(exit_sem, inc=1, device_id=(nbr,),
                                    device_id_type=pl.DeviceIdType.MESH)
            pl.semaphore_wait(exit_sem, 2)

    return pl.pallas_call(
        body,
        out_shape=jax.ShapeDtypeStruct((m, n), jnp.float32),
        in_specs=[pl.BlockSpec(memory_space=pl.ANY),
                  pl.BlockSpec(memory_space=pltpu.VMEM)],
        out_specs=pl.BlockSpec(memory_space=pl.ANY),
        scratch_shapes=[
            pltpu.VMEM((1, ch, k), jnp.float32),
            pltpu.VMEM((N_DEV, ch, nh), jnp.float32),
            pltpu.VMEM((N_DEV, ch, nh), jnp.float32),
            pltpu.VMEM((N_DEV, ch, nh), jnp.int8),
            pltpu.VMEM((N_DEV, ch, nh), jnp.int8),
            pltpu.VMEM((N_DEV, 8, 128), jnp.float32),
            pltpu.SemaphoreType.DMA((3,)),
            pltpu.SemaphoreType.DMA((3,)),
            pltpu.SemaphoreType.DMA((3,)),
            pltpu.SemaphoreType.DMA((3,)),
            pltpu.SemaphoreType.DMA((3,)),
            pltpu.SemaphoreType.DMA((3,)),
            pltpu.SemaphoreType.DMA((3,)),
            pltpu.SemaphoreType.DMA((3,)),
            pltpu.SemaphoreType.DMA((3,)),
            pltpu.SemaphoreType.DMA((3,)),
            pltpu.SemaphoreType.DMA,
            pltpu.SemaphoreType.DMA((N_DEV,)),
            pltpu.SemaphoreType.DMA((N_DEV,)),
        ],
        compiler_params=pltpu.CompilerParams(
            collective_id=0,
            vmem_limit_bytes=60 * 1024 * 1024,
        ),
    )(x, w_mat)


# device time: 228045 ns/iter; 1.5001x vs baseline; 1.0734x over previous
import functools

import jax
import jax.numpy as jnp
from jax import lax
from jax.experimental import pallas as pl
from jax.experimental.pallas import tpu as pltpu

N_DEV = 4


def kernel(x, w_mat):
    m, k = x.shape
    _, n = w_mat.shape
    ch = m // N_DEV
    nh = n // 2

    def body(x_hbm, w_ref, out_hbm, xbuf, buf_a, buf_b, qbuf_a, qbuf_b,
             ptile, amax_buf, rs_send_a, rs_recv_a, rs_send_b, rs_recv_b,
             ag_send_a, ag_recv_a, ag_send_b, ag_recv_b,
             ax_send, ax_recv, xsem, o_a, o_b):
        p = lax.axis_index("i")
        right = jnp.mod(p + 1, N_DEV)
        left = jnp.mod(p - 1, N_DEV)
        cols_a = pl.ds(0, nh)
        cols_b = pl.ds(nh, nh)

        barrier = pltpu.get_barrier_semaphore()
        for nbr in (left, right):
            pl.semaphore_signal(barrier, inc=1, device_id=(nbr,),
                                device_id_type=pl.DeviceIdType.MESH)
        pl.semaphore_wait(barrier, 2)

        def load_x(c):
            cp = pltpu.make_async_copy(
                x_hbm.at[pl.ds(c * ch, ch), :], xbuf.at[0], xsem)
            cp.start()
            return cp

        load_x(p).wait()
        buf_a[0, :, :] = jnp.dot(xbuf[0, :, :], w_ref[:, :nh],
                                 preferred_element_type=jnp.float32)
        buf_b[0, :, :] = jnp.dot(xbuf[0, :, :], w_ref[:, nh:],
                                 preferred_element_type=jnp.float32)

        for s in range(N_DEV - 1):
            rdma_a = pltpu.make_async_remote_copy(
                src_ref=buf_a.at[s], dst_ref=buf_a.at[s + 1],
                send_sem=rs_send_a.at[s], recv_sem=rs_recv_a.at[s],
                device_id=(right,), device_id_type=pl.DeviceIdType.MESH)
            rdma_b = pltpu.make_async_remote_copy(
                src_ref=buf_b.at[s], dst_ref=buf_b.at[s + 1],
                send_sem=rs_send_b.at[s], recv_sem=rs_recv_b.at[s],
                device_id=(left,), device_id_type=pl.DeviceIdType.MESH)
            rdma_a.start()
            rdma_b.start()
            ca = jnp.mod(p - 1 - s, N_DEV)
            cb = jnp.mod(p + 1 + s, N_DEV)
            load_x(ca).wait()
            ptile[0, :, :] = jnp.dot(xbuf[0, :, :], w_ref[:, :nh],
                                     preferred_element_type=jnp.float32)
            if s != 1:
                load_x(cb).wait()
            ptile[1, :, :] = jnp.dot(xbuf[0, :, :], w_ref[:, nh:],
                                     preferred_element_type=jnp.float32)
            rdma_a.wait()
            buf_a[s + 1, :, :] = buf_a[s + 1, :, :] + ptile[0, :, :]
            rdma_b.wait()
            buf_b[s + 1, :, :] = buf_b[s + 1, :, :] + ptile[1, :, :]


        buf_a[3, :, :] = jnp.maximum(buf_a[3, :, :], 0.0)
        buf_b[3, :, :] = jnp.maximum(buf_b[3, :, :], 0.0)
        local_amax = jnp.maximum(jnp.max(buf_a[3, :, :]),
                                 jnp.max(buf_b[3, :, :]))
        amax_buf[pl.ds(p, 1), :, :] = jnp.full((1, 8, 128), local_amax,
                                               jnp.float32)

        ax_rdmas = []
        for j in range(1, N_DEV):
            tgt = jnp.mod(p + j, N_DEV)
            r = pltpu.make_async_remote_copy(
                src_ref=amax_buf.at[pl.ds(p, 1)],
                dst_ref=amax_buf.at[pl.ds(p, 1)],
                send_sem=ax_send.at[j - 1], recv_sem=ax_recv.at[j - 1],
                device_id=(tgt,), device_id_type=pl.DeviceIdType.MESH)
            r.start()
            ax_rdmas.append(r)
        for r in ax_rdmas:
            r.wait()
        amax = jnp.max(amax_buf[:, :, :])
        scale = amax / 127.0

        qbuf_a[0, :, :] = jnp.clip(jnp.round(buf_a[3, :, :] / scale),
                                   -127.0, 127.0).astype(jnp.int8)
        qbuf_b[0, :, :] = jnp.clip(jnp.round(buf_b[3, :, :] / scale),
                                   -127.0, 127.0).astype(jnp.int8)

        out_copies = []

        def stage_out(slot):
            buf_a[slot, :, :] = (qbuf_a[slot, :, :].astype(jnp.float32)
                                 * scale)
            ra = jnp.mod(p + 1 - slot, N_DEV)
            oa = pltpu.make_async_copy(
                buf_a.at[slot],
                out_hbm.at[pl.ds(ra * ch, ch), cols_a], o_a.at[slot])
            oa.start()
            buf_b[slot, :, :] = (qbuf_b[slot, :, :].astype(jnp.float32)
                                 * scale)
            rb = jnp.mod(p - 1 + slot, N_DEV)
            ob = pltpu.make_async_copy(
                buf_b.at[slot],
                out_hbm.at[pl.ds(rb * ch, ch), cols_b], o_b.at[slot])
            ob.start()
            out_copies.extend((oa, ob))

        for s in range(N_DEV - 1):
            rdma_a = pltpu.make_async_remote_copy(
                src_ref=qbuf_a.at[s], dst_ref=qbuf_a.at[s + 1],
                send_sem=ag_send_a.at[s], recv_sem=ag_recv_a.at[s],
                device_id=(right,), device_id_type=pl.DeviceIdType.MESH)
            rdma_b = pltpu.make_async_remote_copy(
                src_ref=qbuf_b.at[s], dst_ref=qbuf_b.at[s + 1],
                send_sem=ag_send_b.at[s], recv_sem=ag_recv_b.at[s],
                device_id=(left,), device_id_type=pl.DeviceIdType.MESH)
            rdma_a.start()
            rdma_b.start()
            stage_out(s)
            rdma_a.wait()
            rdma_b.wait()
        stage_out(N_DEV - 1)

        for cp in out_copies:
            cp.wait()

        @functools.partial(pl.run_scoped,
                           exit_sem=pltpu.SemaphoreType.REGULAR)
        def _(exit_sem):
            for nbr in (left, right):
                pl.semaphore_signal(exit_sem, inc=1, device_id=(nbr,),
                                    device_id_type=pl.DeviceIdType.MESH)
            pl.semaphore_wait(exit_sem, 2)

    return pl.pallas_call(
        body,
        out_shape=jax.ShapeDtypeStruct((m, n), jnp.float32),
        in_specs=[pl.BlockSpec(memory_space=pl.ANY),
                  pl.BlockSpec(memory_space=pltpu.VMEM)],
        out_specs=pl.BlockSpec(memory_space=pl.ANY),
        scratch_shapes=[
            pltpu.VMEM((1, ch, k), jnp.float32),
            pltpu.VMEM((N_DEV, ch, nh), jnp.float32),
            pltpu.VMEM((N_DEV, ch, nh), jnp.float32),
            pltpu.VMEM((N_DEV, ch, nh), jnp.int8),
            pltpu.VMEM((N_DEV, ch, nh), jnp.int8),
            pltpu.VMEM((2, ch, nh), jnp.float32),
            pltpu.VMEM((N_DEV, 8, 128), jnp.float32),
            pltpu.SemaphoreType.DMA((3,)),
            pltpu.SemaphoreType.DMA((3,)),
            pltpu.SemaphoreType.DMA((3,)),
            pltpu.SemaphoreType.DMA((3,)),
            pltpu.SemaphoreType.DMA((3,)),
            pltpu.SemaphoreType.DMA((3,)),
            pltpu.SemaphoreType.DMA((3,)),
            pltpu.SemaphoreType.DMA((3,)),
            pltpu.SemaphoreType.DMA((3,)),
            pltpu.SemaphoreType.DMA((3,)),
            pltpu.SemaphoreType.DMA,
            pltpu.SemaphoreType.DMA((N_DEV,)),
            pltpu.SemaphoreType.DMA((N_DEV,)),
        ],
        compiler_params=pltpu.CompilerParams(
            collective_id=0,
            vmem_limit_bytes=60 * 1024 * 1024,
        ),
    )(x, w_mat)


# device time: 226364 ns/iter; 1.5113x vs baseline; 1.0074x over previous
import functools

import jax
import jax.numpy as jnp
from jax import lax
from jax.experimental import pallas as pl
from jax.experimental.pallas import tpu as pltpu

N_DEV = 4


def kernel(x, w_mat):
    m, k = x.shape
    _, n = w_mat.shape
    ch = m // N_DEV
    nh = n // 2

    def body(x_hbm, w_ref, out_hbm, xbuf, buf_a, buf_b, qbuf_a, qbuf_b,
             ptile, amax_buf, rs_send_a, rs_recv_a, rs_send_b, rs_recv_b,
             ag_send_a, ag_recv_a, ag_send_b, ag_recv_b,
             ax_send, ax_recv, xsem, o_a, o_b):
        p = lax.axis_index("i")
        right = jnp.mod(p + 1, N_DEV)
        left = jnp.mod(p - 1, N_DEV)
        cols_a = pl.ds(0, nh)
        cols_b = pl.ds(nh, nh)

        barrier = pltpu.get_barrier_semaphore()
        for nbr in (left, right):
            pl.semaphore_signal(barrier, inc=1, device_id=(nbr,),
                                device_id_type=pl.DeviceIdType.MESH)
        pl.semaphore_wait(barrier, 2)

        def load_x(c):
            cp = pltpu.make_async_copy(
                x_hbm.at[pl.ds(c * ch, ch), :], xbuf.at[0], xsem)
            cp.start()
            return cp

        def mk_rs(s, half):
            buf, send, recv, tgt = (
                (buf_a, rs_send_a, rs_recv_a, right) if half == 0 else
                (buf_b, rs_send_b, rs_recv_b, left))
            return pltpu.make_async_remote_copy(
                src_ref=buf.at[s], dst_ref=buf.at[s + 1],
                send_sem=send.at[s], recv_sem=recv.at[s],
                device_id=(tgt,), device_id_type=pl.DeviceIdType.MESH)

        def mk_ag(s, half):
            buf, send, recv, tgt = (
                (qbuf_a, ag_send_a, ag_recv_a, right) if half == 0 else
                (qbuf_b, ag_send_b, ag_recv_b, left))
            return pltpu.make_async_remote_copy(
                src_ref=buf.at[s], dst_ref=buf.at[s + 1],
                send_sem=send.at[s], recv_sem=recv.at[s],
                device_id=(tgt,), device_id_type=pl.DeviceIdType.MESH)

        load_x(p).wait()
        buf_a[0, :, :] = jnp.dot(xbuf[0, :, :], w_ref[:, :nh],
                                 preferred_element_type=jnp.float32)
        rd_a = mk_rs(0, 0)
        rd_a.start()
        buf_b[0, :, :] = jnp.dot(xbuf[0, :, :], w_ref[:, nh:],
                                 preferred_element_type=jnp.float32)
        rd_b = mk_rs(0, 1)
        rd_b.start()

        for s in range(N_DEV - 1):
            ca = jnp.mod(p - 1 - s, N_DEV)
            cb = jnp.mod(p + 1 + s, N_DEV)
            load_x(ca).wait()
            ptile[0, :, :] = jnp.dot(xbuf[0, :, :], w_ref[:, :nh],
                                     preferred_element_type=jnp.float32)
            if s != 1:
                load_x(cb).wait()
            ptile[1, :, :] = jnp.dot(xbuf[0, :, :], w_ref[:, nh:],
                                     preferred_element_type=jnp.float32)
            rd_a.wait()
            if s < N_DEV - 2:
                buf_a[s + 1, :, :] = buf_a[s + 1, :, :] + ptile[0, :, :]
                rd_a = mk_rs(s + 1, 0)
                rd_a.start()
            else:
                buf_a[3, :, :] = jnp.maximum(
                    buf_a[3, :, :] + ptile[0, :, :], 0.0)
            rd_b.wait()
            if s < N_DEV - 2:
                buf_b[s + 1, :, :] = buf_b[s + 1, :, :] + ptile[1, :, :]
                rd_b = mk_rs(s + 1, 1)
                rd_b.start()
            else:
                buf_b[3, :, :] = jnp.maximum(
                    buf_b[3, :, :] + ptile[1, :, :], 0.0)

        local_amax = jnp.maximum(jnp.max(buf_a[3, :, :]),
                                 jnp.max(buf_b[3, :, :]))
        amax_buf[pl.ds(p, 1), :, :] = jnp.full((1, 8, 128), local_amax,
                                               jnp.float32)

        ax_rdmas = []
        for j in range(1, N_DEV):
            tgt = jnp.mod(p + j, N_DEV)
            r = pltpu.make_async_remote_copy(
                src_ref=amax_buf.at[pl.ds(p, 1)],
                dst_ref=amax_buf.at[pl.ds(p, 1)],
                send_sem=ax_send.at[j - 1], recv_sem=ax_recv.at[j - 1],
                device_id=(tgt,), device_id_type=pl.DeviceIdType.MESH)
            r.start()
            ax_rdmas.append(r)
        for r in ax_rdmas:
            r.wait()
        amax = jnp.max(amax_buf[:, :, :])
        scale = amax / 127.0

        qbuf_a[0, :, :] = jnp.clip(jnp.round(buf_a[3, :, :] / scale),
                                   -127.0, 127.0).astype(jnp.int8)
        ag_a = mk_ag(0, 0)
        ag_a.start()
        qbuf_b[0, :, :] = jnp.clip(jnp.round(buf_b[3, :, :] / scale),
                                   -127.0, 127.0).astype(jnp.int8)
        ag_b = mk_ag(0, 1)
        ag_b.start()

        out_copies = []

        def stage_out(slot):
            buf_a[slot, :, :] = (qbuf_a[slot, :, :].astype(jnp.float32)
                                 * scale)
            ra = jnp.mod(p + 1 - slot, N_DEV)
            oa = pltpu.make_async_copy(
                buf_a.at[slot],
                out_hbm.at[pl.ds(ra * ch, ch), cols_a], o_a.at[slot])
            oa.start()
            buf_b[slot, :, :] = (qbuf_b[slot, :, :].astype(jnp.float32)
                                 * scale)
            rb = jnp.mod(p - 1 + slot, N_DEV)
            ob = pltpu.make_async_copy(
                buf_b.at[slot],
                out_hbm.at[pl.ds(rb * ch, ch), cols_b], o_b.at[slot])
            ob.start()
            out_copies.extend((oa, ob))

        for s in range(N_DEV - 1):
            stage_out(s)
            ag_a.wait()
            if s < N_DEV - 2:
                ag_a = mk_ag(s + 1, 0)
                ag_a.start()
            ag_b.wait()
            if s < N_DEV - 2:
                ag_b = mk_ag(s + 1, 1)
                ag_b.start()
        stage_out(N_DEV - 1)

        for cp in out_copies:
            cp.wait()

        @functools.partial(pl.run_scoped,
                           exit_sem=pltpu.SemaphoreType.REGULAR)
        def _(exit_sem):
            for nbr in (left, right):
                pl.semaphore_signal(exit_sem, inc=1, device_id=(nbr,),
                                    device_id_type=pl.DeviceIdType.MESH)
            pl.semaphore_wait(exit_sem, 2)

    return pl.pallas_call(
        body,
        out_shape=jax.ShapeDtypeStruct((m, n), jnp.float32),
        in_specs=[pl.BlockSpec(memory_space=pl.ANY),
                  pl.BlockSpec(memory_space=pltpu.VMEM)],
        out_specs=pl.BlockSpec(memory_space=pl.ANY),
        scratch_shapes=[
            pltpu.VMEM((1, ch, k), jnp.float32),
            pltpu.VMEM((N_DEV, ch, nh), jnp.float32),
            pltpu.VMEM((N_DEV, ch, nh), jnp.float32),
            pltpu.VMEM((N_DEV, ch, nh), jnp.int8),
            pltpu.VMEM((N_DEV, ch, nh), jnp.int8),
            pltpu.VMEM((2, ch, nh), jnp.float32),
            pltpu.VMEM((N_DEV, 8, 128), jnp.float32),
            pltpu.SemaphoreType.DMA((3,)),
            pltpu.SemaphoreType.DMA((3,)),
            pltpu.SemaphoreType.DMA((3,)),
            pltpu.SemaphoreType.DMA((3,)),
            pltpu.SemaphoreType.DMA((3,)),
            pltpu.SemaphoreType.DMA((3,)),
            pltpu.SemaphoreType.DMA((3,)),
            pltpu.SemaphoreType.DMA((3,)),
            pltpu.SemaphoreType.DMA((3,)),
            pltpu.SemaphoreType.DMA((3,)),
            pltpu.SemaphoreType.DMA,
            pltpu.SemaphoreType.DMA((N_DEV,)),
            pltpu.SemaphoreType.DMA((N_DEV,)),
        ],
        compiler_params=pltpu.CompilerParams(
            collective_id=0,
            vmem_limit_bytes=60 * 1024 * 1024,
        ),
    )(x, w_mat)
